# baseline (speedup 1.0000x reference)
import functools

import jax
import jax.numpy as jnp
from jax import lax
from jax.experimental import pallas as pl
from jax.experimental.pallas import tpu as pltpu
from jax.experimental.pallas import tpu_sc as plsc

N = 10000
E = 320000
D = 128
G = 64

NC = 2
NS = 16
NW = NC * NS
EPW = E // NW

NP = 10240
NB = 16
RPB = NP // NB
CAP = 1024
CAPB = 1152
ACC_R = RPB + 8
CH = 64

R = 1024
GRID = NP // R

_SC_PARAMS = pltpu.CompilerParams(needs_layout_passes=False,
                                  use_tc_tiling_on_sc=False)


def _mesh():
    return plsc.VectorSubcoreMesh(
        core_axis_name="c", subcore_axis_name="s", num_cores=NC, num_subcores=NS
    )


def _sc_scan(src2, dst2):
    @functools.partial(
        pl.kernel,
        out_type=[
            jax.ShapeDtypeStruct((NW, NP), jnp.float32),
            jax.ShapeDtypeStruct((NW, NB * CAPB), jnp.int32),
            jax.ShapeDtypeStruct((NW, NB * CAPB), jnp.int32),
        ],
        mesh=_mesh(),
        compiler_params=_SC_PARAMS,
        scratch_types=[
            pltpu.VMEM((EPW,), jnp.int32),
            pltpu.VMEM((EPW,), jnp.int32),
            pltpu.VMEM((NP,), jnp.float32),
            pltpu.VMEM((NB * CAPB,), jnp.int32),
            pltpu.VMEM((NB * CAPB,), jnp.int32),
        ],
    )
    def k(src_hbm, dst_hbm, degp_hbm, srcb_hbm, dstb_hbm,
          src_v, dst_v, deg_v, sbuf, dbuf):
        cid = lax.axis_index("c")
        sid = lax.axis_index("s")
        slab = cid * NS + sid
        zeros16 = jnp.zeros((16,), jnp.float32)
        ones16 = jnp.full((16,), 1.0, jnp.float32)
        zi16 = jnp.zeros((16,), jnp.int32)
        dummy16 = jnp.full((16,), RPB, jnp.int32)
        iota16 = lax.iota(jnp.int32, 16)

        pltpu.sync_copy(src_hbm.at[slab], src_v)
        pltpu.sync_copy(dst_hbm.at[slab], dst_v)

        def zdeg(i, carry):
            deg_v[pl.ds(i * 16, 16)] = zeros16
            return carry

        lax.fori_loop(0, NP // 16, zdeg, 0)

        def zbuf(i, carry):
            sbuf[pl.ds(i * 16, 16)] = zi16
            dbuf[pl.ds(i * 16, 16)] = dummy16
            return carry

        lax.fori_loop(0, NB * CAPB // 16, zbuf, 0)

        def body(i, cnts):
            s16 = src_v[pl.ds(i * 16, 16)]
            d16 = dst_v[pl.ds(i * 16, 16)]
            plsc.addupdate_scatter(deg_v, [d16], ones16)
            b16 = lax.shift_right_logical(d16 * 6554, 22)
            dl16 = d16 - b16 * RPB
            new = []
            for b in range(NB):
                m = b16 == b
                cnt = b * CAPB + jnp.minimum(cnts[b], CAP)
                plsc.store_compressed(sbuf.at[pl.ds(cnt, 16)], s16, mask=m)
                plsc.store_compressed(dbuf.at[pl.ds(cnt, 16)], dl16, mask=m)
                new.append(cnts[b] + plsc.all_reduce_population_count(m)[0])
            return tuple(new)

        lax.fori_loop(0, EPW // 16, body, (jnp.int32(0),) * NB)

        pltpu.sync_copy(deg_v, degp_hbm.at[slab])
        pltpu.sync_copy(sbuf, srcb_hbm.at[slab])
        pltpu.sync_copy(dbuf, dstb_hbm.at[slab])

    return k(src2, dst2)


def _sc_accumulate(g, srcb, dstb):
    @functools.partial(
        pl.kernel,
        out_type=jax.ShapeDtypeStruct((NC, NS, RPB, D), jnp.float32),
        mesh=_mesh(),
        compiler_params=_SC_PARAMS,
        scratch_types=[
            pltpu.VMEM((CAP,), jnp.int32),
            pltpu.VMEM((CAP,), jnp.int32),
            pltpu.VMEM((CH, D), jnp.float32),
            pltpu.VMEM((CH, D), jnp.float32),
            pltpu.VMEM((ACC_R, D), jnp.float32),
            pltpu.SemaphoreType.DMA,
            pltpu.SemaphoreType.DMA,
        ],
    )
    def k(g_hbm, srcb_hbm, dstb_hbm, out_hbm,
          src_v, dst_v, bufa, bufb, acc, sema, semb):
        cid = lax.axis_index("c")
        sid = lax.axis_index("s")
        zeros16 = jnp.zeros((16,), jnp.float32)
        zi16 = jnp.zeros((16,), jnp.int32)
        iota16 = lax.iota(jnp.int32, 16)

        @plsc.parallel_loop(0, ACC_R, unroll=4)
        def _zacc(r):
            for j in range(D // 16):
                acc[r, pl.ds(j * 16, 16)] = zeros16

        def accum_chunk(base, rows_ref):
            @plsc.parallel_loop(0, CH, unroll=4)
            def _edge(e):
                dspl = plsc.load_gather(dst_v, [zi16 + (base + e)])
                for j in range(D // 16):
                    plsc.addupdate_scatter(
                        acc, [dspl, j * 16 + iota16],
                        rows_ref[e, pl.ds(j * 16, 16)])

        def region(st, carry):
            slab = cid * NS + st
            roff = pl.multiple_of(sid * CAPB, 128)
            pltpu.sync_copy(srcb_hbm.at[slab, pl.ds(roff, CAP)], src_v)
            pltpu.sync_copy(dstb_hbm.at[slab, pl.ds(roff, CAP)], dst_v)
            pltpu.async_copy(g_hbm.at[src_v.at[pl.ds(0, CH)]], bufa, sema)

            def pair(t, inner):
                ca = 2 * t
                cb = 2 * t + 1
                cn = 2 * t + 2
                pltpu.async_copy(
                    g_hbm.at[src_v.at[pl.ds(cb * CH, CH)]], bufb, semb)
                pltpu.make_async_copy(
                    g_hbm.at[src_v.at[pl.ds(ca * CH, CH)]], bufa, sema).wait()
                accum_chunk(ca * CH, bufa)

                @pl.when(cn < CAP // CH)
                def _():
                    pltpu.async_copy(
                        g_hbm.at[src_v.at[pl.ds(cn * CH, CH)]], bufa, sema)

                pltpu.make_async_copy(
                    g_hbm.at[src_v.at[pl.ds(cb * CH, CH)]], bufb, semb).wait()
                accum_chunk(cb * CH, bufb)
                return inner

            return lax.fori_loop(0, CAP // CH // 2, pair, carry)

        lax.fori_loop(0, NS, region, 0)
        pltpu.sync_copy(acc.at[pl.ds(0, RPB)], out_hbm.at[cid, sid])

    return k(g, srcb, dstb).reshape(NC, NP, D)


def _tc_prep(x_pad, W1, degp):
    def body(x_ref, w_ref, dp_ref, g_ref):
        deg = jnp.sum(dp_ref[...], axis=0) + 1.0
        dinv = lax.rsqrt(deg)
        g_ref[...] = jnp.dot(
            x_ref[...], w_ref[...], preferred_element_type=jnp.float32
        ) * dinv

    return pl.pallas_call(
        body,
        grid=(GRID,),
        in_specs=[
            pl.BlockSpec((R, D), lambda i: (i, 0)),
            pl.BlockSpec((D, D), lambda i: (0, 0)),
            pl.BlockSpec((NW, R, 1), lambda i: (0, i, 0)),
        ],
        out_specs=pl.BlockSpec((R, D), lambda i: (i, 0)),
        out_shape=jax.ShapeDtypeStruct((NP, D), jnp.float32),
    )(x_pad, W1, degp)


def _tc_mid(p0, p1, g1, degp, b1, W2):
    def body(p0_ref, p1_ref, g1_ref, dp_ref, b1_ref, w_ref, g2_ref):
        deg = jnp.sum(dp_ref[...], axis=0) + 1.0
        dinv = lax.rsqrt(deg)
        s = p0_ref[...] + p1_ref[...] + g1_ref[...]
        h = jnp.maximum(s * dinv + b1_ref[...], 0.0)
        g2_ref[...] = jnp.dot(
            h, w_ref[...], preferred_element_type=jnp.float32
        ) * dinv

    return pl.pallas_call(
        body,
        grid=(GRID,),
        in_specs=[
            pl.BlockSpec((R, D), lambda i: (i, 0)),
            pl.BlockSpec((R, D), lambda i: (i, 0)),
            pl.BlockSpec((R, D), lambda i: (i, 0)),
            pl.BlockSpec((NW, R, 1), lambda i: (0, i, 0)),
            pl.BlockSpec((1, D), lambda i: (0, 0)),
            pl.BlockSpec((D, D), lambda i: (0, 0)),
        ],
        out_specs=pl.BlockSpec((R, D), lambda i: (i, 0)),
        out_shape=jax.ShapeDtypeStruct((NP, D), jnp.float32),
    )(p0, p1, g1, degp, b1, W2)


def _tc_final(p0, p1, g2, degp, b2, batch3, Wc1, bc1, Wc2, bc2):
    DH = Wc1.shape[1]
    DO = Wc2.shape[1]

    def body(p0_ref, p1_ref, g2_ref, dp_ref, b2_ref, bt_ref,
             wc1_ref, bc1_ref, wc2_ref, bc2_ref, out_ref, pool_ref, cnt_ref):
        i = pl.program_id(0)

        @pl.when(i == 0)
        def _():
            pool_ref[...] = jnp.zeros_like(pool_ref)
            cnt_ref[...] = jnp.zeros_like(cnt_ref)

        deg = jnp.sum(dp_ref[...], axis=0) + 1.0
        dinv = lax.rsqrt(deg)
        h2 = (p0_ref[...] + p1_ref[...] + g2_ref[...]) * dinv + b2_ref[...]
        bt = bt_ref[...].reshape(1, R)
        ids = lax.broadcasted_iota(jnp.int32, (G, R), 0).astype(jnp.float32)
        onehot = (ids == jnp.broadcast_to(bt, (G, R))).astype(jnp.float32)
        pool_ref[...] += jnp.dot(onehot, h2, preferred_element_type=jnp.float32)
        cnt_ref[...] += jnp.broadcast_to(
            jnp.sum(onehot, axis=1, keepdims=True), (G, D)
        )

        @pl.when(i == GRID - 1)
        def _():
            cnt = jnp.clip(cnt_ref[:, 0:1], 1.0, None)
            pooled = pool_ref[...] / cnt
            z = jnp.maximum(
                jnp.dot(pooled, wc1_ref[...], preferred_element_type=jnp.float32)
                + bc1_ref[...], 0.0)
            out_ref[...] = jnp.dot(
                z, wc2_ref[...], preferred_element_type=jnp.float32
            ) + bc2_ref[...]

    return pl.pallas_call(
        body,
        grid=(GRID,),
        in_specs=[
            pl.BlockSpec((R, D), lambda i: (i, 0)),
            pl.BlockSpec((R, D), lambda i: (i, 0)),
            pl.BlockSpec((R, D), lambda i: (i, 0)),
            pl.BlockSpec((NW, R, 1), lambda i: (0, i, 0)),
            pl.BlockSpec((1, D), lambda i: (0, 0)),
            pl.BlockSpec((1, 1, R), lambda i: (i, 0, 0)),
            pl.BlockSpec((D, DH), lambda i: (0, 0)),
            pl.BlockSpec((1, DH), lambda i: (0, 0)),
            pl.BlockSpec((DH, DO), lambda i: (0, 0)),
            pl.BlockSpec((1, DO), lambda i: (0, 0)),
        ],
        out_specs=pl.BlockSpec((G, DO), lambda i: (0, 0)),
        out_shape=jax.ShapeDtypeStruct((G, DO), jnp.float32),
        scratch_shapes=[
            pltpu.VMEM((G, D), jnp.float32),
            pltpu.VMEM((G, D), jnp.float32),
        ],
    )(p0, p1, g2, degp, b2, batch3, Wc1, bc1, Wc2, bc2)


def kernel(x, edge_index, batch, W1, b1, W2, b2, Wc1, bc1, Wc2, bc2):
    ei = edge_index.astype(jnp.int32)
    src2 = ei[0].reshape(NW, EPW)
    dst2 = ei[1].reshape(NW, EPW)

    degp, srcb, dstb = _sc_scan(src2, dst2)
    degp3 = degp.reshape(NW, NP, 1)

    x_pad = jnp.pad(x, ((0, NP - N), (0, 0)))
    g1 = _tc_prep(x_pad, W1, degp3)
    parts1 = _sc_accumulate(g1, srcb, dstb)
    g2 = _tc_mid(parts1[0], parts1[1], g1, degp3,
                 b1.reshape(1, D).astype(jnp.float32), W2)
    parts2 = _sc_accumulate(g2, srcb, dstb)

    batch3 = jnp.concatenate(
        [batch.astype(jnp.float32), jnp.full((NP - N,), float(G), jnp.float32)]
    ).reshape(GRID, 1, R)
    out = _tc_final(parts2[0], parts2[1], g2, degp3,
                    b2.reshape(1, D).astype(jnp.float32), batch3,
                    Wc1, bc1.reshape(1, -1), Wc2, bc2.reshape(1, -1))
    return out


# speedup vs baseline: 17.0519x; 17.0519x over previous
import functools

import jax
import jax.numpy as jnp
from jax import lax
from jax.experimental import pallas as pl
from jax.experimental.pallas import tpu as pltpu
from jax.experimental.pallas import tpu_sc as plsc

N = 10000
E = 320000
D = 128
G = 64

NC = 2
NS = 16
NW = NC * NS
EPW = E // NW

NP = 10240
NB = 16
RPB = NP // NB
CAP = 1024
CAPB = 1152
ACC_R = RPB + 8
CH = 64

R = 1024
GRID = NP // R

_SC_PARAMS = pltpu.CompilerParams(needs_layout_passes=False,
                                  use_tc_tiling_on_sc=False)


def _mesh():
    return plsc.VectorSubcoreMesh(
        core_axis_name="c", subcore_axis_name="s", num_cores=NC, num_subcores=NS
    )


def _sc_scan(src2, dst2):
    @functools.partial(
        pl.kernel,
        out_type=[
            jax.ShapeDtypeStruct((NW, NP), jnp.float32),
            jax.ShapeDtypeStruct((NW, NB * CAPB), jnp.int32),
            jax.ShapeDtypeStruct((NW, NB * CAPB), jnp.int32),
        ],
        mesh=_mesh(),
        compiler_params=_SC_PARAMS,
        scratch_types=[
            pltpu.VMEM((EPW,), jnp.int32),
            pltpu.VMEM((EPW,), jnp.int32),
            pltpu.VMEM((NP,), jnp.float32),
            pltpu.VMEM((NB * CAPB,), jnp.int32),
            pltpu.VMEM((NB * CAPB,), jnp.int32),
        ],
    )
    def k(src_hbm, dst_hbm, degp_hbm, srcb_hbm, dstb_hbm,
          src_v, dst_v, deg_v, sbuf, dbuf):
        cid = lax.axis_index("c")
        sid = lax.axis_index("s")
        slab = cid * NS + sid
        zeros16 = jnp.zeros((16,), jnp.float32)
        ones16 = jnp.full((16,), 1.0, jnp.float32)
        zi16 = jnp.zeros((16,), jnp.int32)
        dummy16 = jnp.full((16,), RPB, jnp.int32)
        iota16 = lax.iota(jnp.int32, 16)

        pltpu.sync_copy(src_hbm.at[slab], src_v)
        pltpu.sync_copy(dst_hbm.at[slab], dst_v)

        def zdeg(i, carry):
            deg_v[pl.ds(i * 16, 16)] = zeros16
            return carry

        lax.fori_loop(0, NP // 16, zdeg, 0)

        def zbuf(i, carry):
            sbuf[pl.ds(i * 16, 16)] = (zi16 + i * 16 + iota16) & 8191
            dbuf[pl.ds(i * 16, 16)] = dummy16
            return carry

        lax.fori_loop(0, NB * CAPB // 16, zbuf, 0)

        def body(i, cnts):
            s16 = src_v[pl.ds(i * 16, 16)]
            d16 = dst_v[pl.ds(i * 16, 16)]
            plsc.addupdate_scatter(deg_v, [d16], ones16)
            b16 = lax.shift_right_logical(d16 * 6554, 22)
            dl16 = d16 - b16 * RPB
            new = []
            for b in range(NB):
                m = b16 == b
                cnt = b * CAPB + jnp.minimum(cnts[b], CAP)
                plsc.store_compressed(sbuf.at[pl.ds(cnt, 16)], s16, mask=m)
                plsc.store_compressed(dbuf.at[pl.ds(cnt, 16)], dl16, mask=m)
                new.append(cnts[b] + plsc.all_reduce_population_count(m)[0])
            return tuple(new)

        lax.fori_loop(0, EPW // 16, body, (jnp.int32(0),) * NB)

        pltpu.sync_copy(deg_v, degp_hbm.at[slab])
        pltpu.sync_copy(sbuf, srcb_hbm.at[slab])
        pltpu.sync_copy(dbuf, dstb_hbm.at[slab])

    return k(src2, dst2)


def _sc_accumulate(g, srcb, dstb):
    @functools.partial(
        pl.kernel,
        out_type=jax.ShapeDtypeStruct((NC, NS, RPB, D), jnp.float32),
        mesh=_mesh(),
        compiler_params=_SC_PARAMS,
        scratch_types=[
            pltpu.VMEM((CAP,), jnp.int32),
            pltpu.VMEM((CAP,), jnp.int32),
            pltpu.VMEM((CH, D), jnp.float32),
            pltpu.VMEM((CH, D), jnp.float32),
            pltpu.VMEM((ACC_R, D), jnp.float32),
            pltpu.SemaphoreType.DMA,
            pltpu.SemaphoreType.DMA,
        ],
    )
    def k(g_hbm, srcb_hbm, dstb_hbm, out_hbm,
          src_v, dst_v, bufa, bufb, acc, sema, semb):
        cid = lax.axis_index("c")
        sid = lax.axis_index("s")
        zeros16 = jnp.zeros((16,), jnp.float32)
        zi16 = jnp.zeros((16,), jnp.int32)
        iota16 = lax.iota(jnp.int32, 16)

        @plsc.parallel_loop(0, ACC_R, unroll=4)
        def _zacc(r):
            for j in range(D // 16):
                acc[r, pl.ds(j * 16, 16)] = zeros16

        def accum_chunk(base, rows_ref):
            @plsc.parallel_loop(0, CH, unroll=4)
            def _edge(e):
                dspl = plsc.load_gather(dst_v, [zi16 + (base + e)])
                for j in range(D // 16):
                    plsc.addupdate_scatter(
                        acc, [dspl, j * 16 + iota16],
                        rows_ref[e, pl.ds(j * 16, 16)])

        def region(st, carry):
            slab = cid * NS + st
            roff = pl.multiple_of(sid * CAPB, 128)
            pltpu.sync_copy(srcb_hbm.at[slab, pl.ds(roff, CAP)], src_v)
            pltpu.sync_copy(dstb_hbm.at[slab, pl.ds(roff, CAP)], dst_v)
            pltpu.async_copy(g_hbm.at[src_v.at[pl.ds(0, CH)]], bufa, sema)

            def pair(t, inner):
                ca = 2 * t
                cb = 2 * t + 1
                cn = 2 * t + 2
                pltpu.async_copy(
                    g_hbm.at[src_v.at[pl.ds(cb * CH, CH)]], bufb, semb)
                pltpu.make_async_copy(
                    g_hbm.at[src_v.at[pl.ds(ca * CH, CH)]], bufa, sema).wait()
                accum_chunk(ca * CH, bufa)

                @pl.when(cn < CAP // CH)
                def _():
                    pltpu.async_copy(
                        g_hbm.at[src_v.at[pl.ds(cn * CH, CH)]], bufa, sema)

                pltpu.make_async_copy(
                    g_hbm.at[src_v.at[pl.ds(cb * CH, CH)]], bufb, semb).wait()
                accum_chunk(cb * CH, bufb)
                return inner

            return lax.fori_loop(0, CAP // CH // 2, pair, carry)

        lax.fori_loop(0, NS, region, 0)
        pltpu.sync_copy(acc.at[pl.ds(0, RPB)], out_hbm.at[cid, sid])

    return k(g, srcb, dstb).reshape(NC, NP, D)


def _tc_prep(x_pad, W1, degp):
    def body(x_ref, w_ref, dp_ref, g_ref):
        deg = jnp.sum(dp_ref[...], axis=0) + 1.0
        dinv = lax.rsqrt(deg)
        g_ref[...] = jnp.dot(
            x_ref[...], w_ref[...], preferred_element_type=jnp.float32
        ) * dinv

    return pl.pallas_call(
        body,
        grid=(GRID,),
        in_specs=[
            pl.BlockSpec((R, D), lambda i: (i, 0)),
            pl.BlockSpec((D, D), lambda i: (0, 0)),
            pl.BlockSpec((NW, R, 1), lambda i: (0, i, 0)),
        ],
        out_specs=pl.BlockSpec((R, D), lambda i: (i, 0)),
        out_shape=jax.ShapeDtypeStruct((NP, D), jnp.float32),
    )(x_pad, W1, degp)


def _tc_mid(p0, p1, g1, degp, b1, W2):
    def body(p0_ref, p1_ref, g1_ref, dp_ref, b1_ref, w_ref, g2_ref):
        deg = jnp.sum(dp_ref[...], axis=0) + 1.0
        dinv = lax.rsqrt(deg)
        s = p0_ref[...] + p1_ref[...] + g1_ref[...]
        h = jnp.maximum(s * dinv + b1_ref[...], 0.0)
        g2_ref[...] = jnp.dot(
            h, w_ref[...], preferred_element_type=jnp.float32
        ) * dinv

    return pl.pallas_call(
        body,
        grid=(GRID,),
        in_specs=[
            pl.BlockSpec((R, D), lambda i: (i, 0)),
            pl.BlockSpec((R, D), lambda i: (i, 0)),
            pl.BlockSpec((R, D), lambda i: (i, 0)),
            pl.BlockSpec((NW, R, 1), lambda i: (0, i, 0)),
            pl.BlockSpec((1, D), lambda i: (0, 0)),
            pl.BlockSpec((D, D), lambda i: (0, 0)),
        ],
        out_specs=pl.BlockSpec((R, D), lambda i: (i, 0)),
        out_shape=jax.ShapeDtypeStruct((NP, D), jnp.float32),
    )(p0, p1, g1, degp, b1, W2)


def _tc_final(p0, p1, g2, degp, b2, batch3, Wc1, bc1, Wc2, bc2):
    DH = Wc1.shape[1]
    DO = Wc2.shape[1]

    def body(p0_ref, p1_ref, g2_ref, dp_ref, b2_ref, bt_ref,
             wc1_ref, bc1_ref, wc2_ref, bc2_ref, out_ref, pool_ref, cnt_ref):
        i = pl.program_id(0)

        @pl.when(i == 0)
        def _():
            pool_ref[...] = jnp.zeros_like(pool_ref)
            cnt_ref[...] = jnp.zeros_like(cnt_ref)

        deg = jnp.sum(dp_ref[...], axis=0) + 1.0
        dinv = lax.rsqrt(deg)
        h2 = (p0_ref[...] + p1_ref[...] + g2_ref[...]) * dinv + b2_ref[...]
        bt = bt_ref[...].reshape(1, R)
        ids = lax.broadcasted_iota(jnp.int32, (G, R), 0).astype(jnp.float32)
        onehot = (ids == jnp.broadcast_to(bt, (G, R))).astype(jnp.float32)
        pool_ref[...] += jnp.dot(onehot, h2, preferred_element_type=jnp.float32)
        cnt_ref[...] += jnp.broadcast_to(
            jnp.sum(onehot, axis=1, keepdims=True), (G, D)
        )

        @pl.when(i == GRID - 1)
        def _():
            cnt = jnp.clip(cnt_ref[:, 0:1], 1.0, None)
            pooled = pool_ref[...] / cnt
            z = jnp.maximum(
                jnp.dot(pooled, wc1_ref[...], preferred_element_type=jnp.float32)
                + bc1_ref[...], 0.0)
            out_ref[...] = jnp.dot(
                z, wc2_ref[...], preferred_element_type=jnp.float32
            ) + bc2_ref[...]

    return pl.pallas_call(
        body,
        grid=(GRID,),
        in_specs=[
            pl.BlockSpec((R, D), lambda i: (i, 0)),
            pl.BlockSpec((R, D), lambda i: (i, 0)),
            pl.BlockSpec((R, D), lambda i: (i, 0)),
            pl.BlockSpec((NW, R, 1), lambda i: (0, i, 0)),
            pl.BlockSpec((1, D), lambda i: (0, 0)),
            pl.BlockSpec((1, 1, R), lambda i: (i, 0, 0)),
            pl.BlockSpec((D, DH), lambda i: (0, 0)),
            pl.BlockSpec((1, DH), lambda i: (0, 0)),
            pl.BlockSpec((DH, DO), lambda i: (0, 0)),
            pl.BlockSpec((1, DO), lambda i: (0, 0)),
        ],
        out_specs=pl.BlockSpec((G, DO), lambda i: (0, 0)),
        out_shape=jax.ShapeDtypeStruct((G, DO), jnp.float32),
        scratch_shapes=[
            pltpu.VMEM((G, D), jnp.float32),
            pltpu.VMEM((G, D), jnp.float32),
        ],
    )(p0, p1, g2, degp, b2, batch3, Wc1, bc1, Wc2, bc2)


def kernel(x, edge_index, batch, W1, b1, W2, b2, Wc1, bc1, Wc2, bc2):
    ei = edge_index.astype(jnp.int32)
    src2 = ei[0].reshape(NW, EPW)
    dst2 = ei[1].reshape(NW, EPW)

    degp, srcb, dstb = _sc_scan(src2, dst2)
    degp3 = degp.reshape(NW, NP, 1)

    x_pad = jnp.pad(x, ((0, NP - N), (0, 0)))
    g1 = _tc_prep(x_pad, W1, degp3)
    parts1 = _sc_accumulate(g1, srcb, dstb)
    g2 = _tc_mid(parts1[0], parts1[1], g1, degp3,
                 b1.reshape(1, D).astype(jnp.float32), W2)
    parts2 = _sc_accumulate(g2, srcb, dstb)

    batch3 = jnp.concatenate(
        [batch.astype(jnp.float32), jnp.full((NP - N,), float(G), jnp.float32)]
    ).reshape(GRID, 1, R)
    out = _tc_final(parts2[0], parts2[1], g2, degp3,
                    b2.reshape(1, D).astype(jnp.float32), batch3,
                    Wc1, bc1.reshape(1, -1), Wc2, bc2.reshape(1, -1))
    return out
